# baseline (speedup 1.0000x reference)
import functools

import jax
import jax.numpy as jnp
from jax import lax
from jax.experimental import pallas as pl
from jax.experimental.pallas import tpu as pltpu
from jax.experimental.pallas import tpu_sc as plsc

_NC = 2
_NS = 16
_NW = _NC * _NS
_CH = 16
_U = 8
_K = 4
_P = 2


def _sc_add(x_flat, pos_flat, B, S, D):
    rows_per_w = S // _NW
    chunks = rows_per_w // _CH
    CE = _CH * D
    NJ = chunks * B

    mesh = plsc.VectorSubcoreMesh(core_axis_name="c", subcore_axis_name="s")

    @functools.partial(
        pl.kernel,
        mesh=mesh,
        out_type=jax.ShapeDtypeStruct((B * S * D,), jnp.float32),
        scratch_types=(
            [pltpu.VMEM((CE,), jnp.float32) for _ in range(2 + _K)]
            + [pltpu.SemaphoreType.DMA for _ in range(2 + 2 * _K)]
        ),
    )
    def k(x_hbm, pos_hbm, out_hbm, *scratch):
        pos_bufs = scratch[0:2]
        x_bufs = scratch[2:2 + _K]
        pos_sems = scratch[2 + _K:4 + _K]
        in_sems = scratch[4 + _K:4 + 2 * _K]
        out_sems = scratch[4 + 2 * _K:4 + 3 * _K]

        wid = lax.axis_index("s") * _NC + lax.axis_index("c")
        base_row = wid * rows_per_w

        def pos_slice(c):
            off = pl.multiple_of((base_row + c * _CH) * D, CE)
            return pos_hbm.at[pl.ds(off, CE)]

        def x_slice(hbm, c, b):
            off = pl.multiple_of((b * S + base_row + c * _CH) * D, CE)
            return hbm.at[pl.ds(off, CE)]

        pos_desc = {0: pltpu.async_copy(pos_slice(0), pos_bufs[0], pos_sems[0])}
        in_desc = {}
        out_desc = {}
        out_waited = set()
        for j in range(min(_P, NJ)):
            c, b = divmod(j, B)
            in_desc[j] = pltpu.async_copy(
                x_slice(x_hbm, c, b), x_bufs[j % _K], in_sems[j % _K])

        for j in range(NJ):
            c, b = divmod(j, B)
            if b == 0:
                if c + 1 < chunks:
                    pos_desc[c + 1] = pltpu.async_copy(
                        pos_slice(c + 1), pos_bufs[(c + 1) % 2],
                        pos_sems[(c + 1) % 2])
                pos_desc[c].wait()
            nj = j + _P
            if nj < NJ:
                prev = nj - _K
                if prev >= 0:
                    out_desc[prev].wait()
                    out_waited.add(prev)
                nc, nb = divmod(nj, B)
                in_desc[nj] = pltpu.async_copy(
                    x_slice(x_hbm, nc, nb), x_bufs[nj % _K], in_sems[nj % _K])
            in_desc[j].wait()

            xv = x_bufs[j % _K]
            pv = pos_bufs[c % 2]

            @plsc.parallel_loop(0, CE, step=16, unroll=_U)
            def add_u(s, xv=xv, pv=pv):
                sl = pl.ds(s, 16)
                xv[sl] = xv[sl] + pv[sl]
            out_desc[j] = pltpu.async_copy(
                xv, x_slice(out_hbm, c, b), out_sems[j % _K])

        for j in range(NJ):
            if j not in out_waited:
                out_desc[j].wait()

    return k(x_flat, pos_flat)


def kernel(x, pos_table):
    B, S, D = x.shape
    out_flat = _sc_add(x.reshape(-1), pos_table[:S].reshape(-1), B, S, D)
    return out_flat.reshape(B, S, D)


# speedup vs baseline: 2.7852x; 2.7852x over previous
import functools

import jax
import jax.numpy as jnp
from jax import lax
from jax.experimental import pallas as pl
from jax.experimental.pallas import tpu as pltpu
from jax.experimental.pallas import tpu_sc as plsc

_NC = 2
_NS = 16
_NW = _NC * _NS
_CH = 16
_U = 8
_K = 4
_P = 2


def _sc_add(x2, pos2, B, S, D):
    rows_per_w = S // _NW
    chunks = rows_per_w // _CH
    NJ = chunks * B
    VECS = (_CH * D) // 16

    mesh = plsc.VectorSubcoreMesh(core_axis_name="c", subcore_axis_name="s")

    @functools.partial(
        pl.kernel,
        mesh=mesh,
        out_type=jax.ShapeDtypeStruct((B * S, D), jnp.float32),
        scratch_types=(
            [pltpu.VMEM((_CH, D), jnp.float32) for _ in range(2 + _K)]
            + [pltpu.SemaphoreType.DMA for _ in range(2 + 2 * _K)]
        ),
    )
    def k(x_hbm, pos_hbm, out_hbm, *scratch):
        pos_bufs = scratch[0:2]
        x_bufs = scratch[2:2 + _K]
        pos_sems = scratch[2 + _K:4 + _K]
        in_sems = scratch[4 + _K:4 + 2 * _K]
        out_sems = scratch[4 + 2 * _K:4 + 3 * _K]

        wid = lax.axis_index("s") * _NC + lax.axis_index("c")
        base_row = wid * rows_per_w

        def pos_slice(c):
            return pos_hbm.at[pl.ds(pl.multiple_of(base_row + c * _CH, _CH), _CH), :]

        def x_slice(hbm, c, b):
            row = pl.multiple_of(b * S + base_row + c * _CH, _CH)
            return hbm.at[pl.ds(row, _CH), :]

        pos_desc = {0: pltpu.async_copy(pos_slice(0), pos_bufs[0], pos_sems[0])}
        in_desc = {}
        out_desc = {}
        out_waited = set()
        for j in range(min(_P, NJ)):
            c, b = divmod(j, B)
            in_desc[j] = pltpu.async_copy(
                x_slice(x_hbm, c, b), x_bufs[j % _K], in_sems[j % _K])

        for j in range(NJ):
            c, b = divmod(j, B)
            if b == 0:
                if c + 1 < chunks:
                    pos_desc[c + 1] = pltpu.async_copy(
                        pos_slice(c + 1), pos_bufs[(c + 1) % 2],
                        pos_sems[(c + 1) % 2])
                pos_desc[c].wait()
            nj = j + _P
            if nj < NJ:
                prev = nj - _K
                if prev >= 0:
                    out_desc[prev].wait()
                    out_waited.add(prev)
                nc, nb = divmod(nj, B)
                in_desc[nj] = pltpu.async_copy(
                    x_slice(x_hbm, nc, nb), x_bufs[nj % _K], in_sems[nj % _K])
            in_desc[j].wait()

            xv = x_bufs[j % _K]
            pv = pos_bufs[c % 2]

            @plsc.parallel_loop(0, VECS, step=1, unroll=_U)
            def add_u(i, xv=xv, pv=pv):
                r = i >> 6
                sl = pl.ds((i & 63) * 16, 16)
                xv[r, sl] = xv[r, sl] + pv[r, sl]

            out_desc[j] = pltpu.async_copy(
                xv, x_slice(out_hbm, c, b), out_sems[j % _K])

        for j in range(NJ):
            if j not in out_waited:
                out_desc[j].wait()

    return k(x2, pos2)


def kernel(x, pos_table):
    B, S, D = x.shape
    out2 = _sc_add(x.reshape(B * S, D), pos_table[:S], B, S, D)
    return out2.reshape(B, S, D)
